# baseline (speedup 1.0000x reference)
import functools

import jax
import jax.numpy as jnp
from jax import lax
from jax.experimental import pallas as pl
from jax.experimental.pallas import tpu as pltpu
from jax.experimental.pallas import tpu_sc as plsc

NC = 2
NS = 16
NW = NC * NS
L = 16


def _rsqrt(x):
    i = plsc.bitcast(x, jnp.int32)
    i = jnp.int32(0x5F3759DF) - lax.shift_right_logical(i, 1)
    y = plsc.bitcast(i, jnp.float32)
    xh = x * 0.5
    for _ in range(3):
        y = y * (1.5 - xh * y * y)
    return y


def _make_kernel(H, V, T, B):
    HW = H // NW
    NCHUNK = HW // B
    G = B // L
    NP = B // 128
    NV = 3 * B // 128
    assert H % NW == 0 and HW % B == 0 and B % 128 == 0

    mesh = plsc.VectorSubcoreMesh(core_axis_name="c", subcore_axis_name="s",
                                  num_cores=NC, num_subcores=NS)

    @functools.partial(
        pl.kernel,
        out_type=jax.ShapeDtypeStruct((H, 3), jnp.float32),
        mesh=mesh,
        compiler_params=pltpu.CompilerParams(needs_layout_passes=False,
                                             use_tc_tiling_on_sc=False),
        scratch_types=[
            pltpu.VMEM((NP, 128), jnp.int32),
            pltpu.VMEM((NP, 128), jnp.int32),
            pltpu.VMEM((B, 8), jnp.int32),
            pltpu.VMEM((NV, 128), jnp.int32),
            pltpu.VMEM((3 * B, 8), jnp.float32),
            pltpu.VMEM((B, 3), jnp.float32),
            pltpu.VMEM((B, 3), jnp.float32),
            pltpu.VMEM((B, 3), jnp.float32),
            pltpu.SemaphoreType.DMA,
            pltpu.SemaphoreType.DMA,
        ],
    )
    def vertex_kernel(hp_hbm, hn_hbm, prim_hbm, ibo2_hbm, comb_hbm, out_hbm,
                      prim_v, pidx_v, tri_v, vidx_v, vert_v, hp_v, hn_v, out_v,
                      lsem, gsem):
        wid = lax.axis_index("s") * NC + lax.axis_index("c")
        base0 = wid * HW
        iota = lax.broadcasted_iota(jnp.int32, (L,), 0)

        col = [jnp.full((L,), c, jnp.int32) for c in range(8)]

        def chunk_body(ci, carry):
            base = base0 + ci * B

            descs = []
            for r in range(NP):
                descs.append(pltpu.async_copy(
                    prim_hbm.at[pl.ds(base + 128 * r, 128)],
                    prim_v.at[r], lsem))
            descs.append(pltpu.async_copy(hp_hbm.at[pl.ds(base, B)], hp_v,
                                          lsem))
            descs.append(pltpu.async_copy(hn_hbm.at[pl.ds(base, B)], hn_v,
                                          lsem))
            for d in descs:
                d.wait()

            for r in range(NP):
                for k in range(128 // L):
                    p = prim_v[r, pl.ds(L * k, L)]
                    pidx_v[r, pl.ds(L * k, L)] = lax.shift_right_logical(p, 1)
            descs = []
            for r in range(NP):
                descs.append(pltpu.async_copy(
                    ibo2_hbm.at[pidx_v.at[r]],
                    tri_v.at[pl.ds(128 * r, 128)], gsem))
            for d in descs:
                d.wait()

            def repack(g, c2):
                h16 = g * L + iota
                prow = jnp.full((L,), 0, jnp.int32) + lax.div(g, 8)
                pcol = (lax.rem(g, 8) * L) + iota
                p16 = plsc.load_gather(prim_v, [prow, pcol])
                off = (p16 & 1) * 3
                for s in range(3):
                    vals = plsc.load_gather(tri_v, [h16, off + s])
                    p0 = s * B + g * L
                    row = jnp.full((L,), 0, jnp.int32) + lax.div(p0, 128)
                    c0 = lax.rem(p0, 128) + iota
                    plsc.store_scatter(vidx_v, [row, c0], vals)
                return c2
            lax.fori_loop(0, G, repack, 0, unroll=2)

            descs = []
            for r in range(NV):
                descs.append(pltpu.async_copy(
                    comb_hbm.at[vidx_v.at[r]],
                    vert_v.at[pl.ds(128 * r, 128)], gsem))
            for d in descs:
                d.wait()

            def compute(g, c2):
                h16 = g * L + iota
                hb = h16 + B
                hc = h16 + 2 * B
                ax = plsc.load_gather(vert_v, [h16, col[0]])
                ay = plsc.load_gather(vert_v, [h16, col[1]])
                az = plsc.load_gather(vert_v, [h16, col[2]])
                bx = plsc.load_gather(vert_v, [hb, col[0]])
                by = plsc.load_gather(vert_v, [hb, col[1]])
                bz = plsc.load_gather(vert_v, [hb, col[2]])
                cx = plsc.load_gather(vert_v, [hc, col[0]])
                cy = plsc.load_gather(vert_v, [hc, col[1]])
                cz = plsc.load_gather(vert_v, [hc, col[2]])
                px = plsc.load_gather(hp_v, [h16, col[0]])
                py = plsc.load_gather(hp_v, [h16, col[1]])
                pz = plsc.load_gather(hp_v, [h16, col[2]])

                v0x, v0y, v0z = bx - ax, by - ay, bz - az
                v1x, v1y, v1z = cx - ax, cy - ay, cz - az
                v2x, v2y, v2z = px - ax, py - ay, pz - az
                d00 = v0x * v0x + v0y * v0y + v0z * v0z
                d01 = v0x * v1x + v0y * v1y + v0z * v1z
                d11 = v1x * v1x + v1y * v1y + v1z * v1z
                d20 = v2x * v0x + v2y * v0y + v2z * v0z
                d21 = v2x * v1x + v2y * v1y + v2z * v1z
                denom = d00 * d11 - d01 * d01
                denom = jnp.where(jnp.abs(denom) < 1e-12, 1e-12, denom)
                rden = 1.0 / denom
                v = (d11 * d20 - d01 * d21) * rden
                w = (d00 * d21 - d01 * d20) * rden
                u = 1.0 - v - w

                fax = plsc.load_gather(vert_v, [h16, col[3]])
                fay = plsc.load_gather(vert_v, [h16, col[4]])
                faz = plsc.load_gather(vert_v, [h16, col[5]])
                fbx = plsc.load_gather(vert_v, [hb, col[3]])
                fby = plsc.load_gather(vert_v, [hb, col[4]])
                fbz = plsc.load_gather(vert_v, [hb, col[5]])
                fcx = plsc.load_gather(vert_v, [hc, col[3]])
                fcy = plsc.load_gather(vert_v, [hc, col[4]])
                fcz = plsc.load_gather(vert_v, [hc, col[5]])
                nmx = u * fax + v * fbx + w * fcx
                nmy = u * fay + v * fby + w * fcy
                nmz = u * faz + v * fbz + w * fcz

                nx = plsc.load_gather(hn_v, [h16, col[0]]) + (nmx * 0.5 - 0.25)
                ny = plsc.load_gather(hn_v, [h16, col[1]]) + (nmy * 0.5 - 0.25)
                nz = plsc.load_gather(hn_v, [h16, col[2]]) + (nmz * 0.5 - 0.25)
                rn = _rsqrt(nx * nx + ny * ny + nz * nz)
                plsc.store_scatter(out_v, [h16, col[0]], nx * rn)
                plsc.store_scatter(out_v, [h16, col[1]], ny * rn)
                plsc.store_scatter(out_v, [h16, col[2]], nz * rn)
                return c2
            lax.fori_loop(0, G, compute, 0, unroll=2)

            pltpu.sync_copy(out_v, out_hbm.at[pl.ds(base, B)])
            return carry

        lax.fori_loop(0, NCHUNK, chunk_body, 0)

    return vertex_kernel


def kernel(hit_positions, hit_normals, hit_primIDs, vbo, ibo, features):
    H = hit_primIDs.shape[0]
    V = vbo.shape[0]
    T = ibo.shape[0]
    ibo2 = jnp.concatenate(
        [ibo.reshape(T // 2, 6), jnp.zeros((T // 2, 2), jnp.int32)], axis=1)
    comb = jnp.concatenate(
        [vbo, features, jnp.zeros((V, 2), jnp.float32)], axis=1)
    k = _make_kernel(H, V, T, B=512)
    return k(hit_positions, hit_normals, hit_primIDs, ibo2, comb)


# speedup vs baseline: 13.4076x; 13.4076x over previous
import functools

import jax
import jax.numpy as jnp
from jax import lax
from jax.experimental import pallas as pl
from jax.experimental.pallas import tpu as pltpu
from jax.experimental.pallas import tpu_sc as plsc

NC = 2
NS = 16
NW = NC * NS
L = 16


def _rsqrt(x):
    i = plsc.bitcast(x, jnp.int32)
    i = jnp.int32(0x5F3759DF) - lax.shift_right_logical(i, 1)
    y = plsc.bitcast(i, jnp.float32)
    xh = x * 0.5
    for _ in range(3):
        y = y * (1.5 - xh * y * y)
    return y


def _make_interleave(VP, BV):
    VW = VP // NW
    NCH = VW // BV
    G = BV // L
    assert VP % NW == 0 and VW % BV == 0 and BV % L == 0

    mesh = plsc.VectorSubcoreMesh(core_axis_name="c", subcore_axis_name="s",
                                  num_cores=NC, num_subcores=NS)

    @functools.partial(
        pl.kernel,
        out_type=jax.ShapeDtypeStruct((VP, 8), jnp.float32),
        mesh=mesh,
        compiler_params=pltpu.CompilerParams(needs_layout_passes=False,
                                             use_tc_tiling_on_sc=False),
        scratch_types=[
            pltpu.VMEM((BV,), jnp.float32),
            pltpu.VMEM((BV,), jnp.float32),
            pltpu.VMEM((BV,), jnp.float32),
            pltpu.VMEM((BV,), jnp.float32),
            pltpu.VMEM((BV,), jnp.float32),
            pltpu.VMEM((BV,), jnp.float32),
            pltpu.VMEM((BV, 8), jnp.float32),
            pltpu.SemaphoreType.DMA,
        ],
    )
    def interleave_kernel(c0, c1, c2, c3, c4, c5, comb_hbm,
                          v0, v1, v2, v3, v4, v5, cmb_v, lsem):
        wid = lax.axis_index("s") * NC + lax.axis_index("c")
        base0 = wid * VW
        iota = lax.broadcasted_iota(jnp.int32, (L,), 0)
        col = [jnp.full((L,), c, jnp.int32) for c in range(8)]
        zero = jnp.zeros((L,), jnp.float32)

        def zinit(g, c2_):
            n16 = g * L + iota
            plsc.store_scatter(cmb_v, [n16, col[6]], zero)
            plsc.store_scatter(cmb_v, [n16, col[7]], zero)
            return c2_
        lax.fori_loop(0, G, zinit, 0, unroll=2)

        def chunk_body(ci, carry):
            base = base0 + ci * BV
            descs = []
            for src, dst in ((c0, v0), (c1, v1), (c2, v2),
                             (c3, v3), (c4, v4), (c5, v5)):
                descs.append(pltpu.async_copy(src.at[pl.ds(base, BV)], dst,
                                              lsem))
            for d in descs:
                d.wait()

            def interleave(g, c2_):
                n16 = g * L + iota
                for c, v in ((0, v0), (1, v1), (2, v2),
                             (3, v3), (4, v4), (5, v5)):
                    vals = plsc.load_gather(v, [n16])
                    plsc.store_scatter(cmb_v, [n16, col[c]], vals)
                return c2_
            lax.fori_loop(0, G, interleave, 0, unroll=2)

            pltpu.sync_copy(cmb_v, comb_hbm.at[pl.ds(base, BV)])
            return carry

        lax.fori_loop(0, NCH, chunk_body, 0)

    return interleave_kernel


def _make_kernel(H, V, T, B):
    HW = H // NW
    NCHUNK = HW // B
    G = B // L
    NP = B // 128
    NV = 3 * B // 128
    assert H % NW == 0 and HW % B == 0 and B % 128 == 0

    mesh = plsc.VectorSubcoreMesh(core_axis_name="c", subcore_axis_name="s",
                                  num_cores=NC, num_subcores=NS)

    @functools.partial(
        pl.kernel,
        out_type=tuple(jax.ShapeDtypeStruct((H,), jnp.float32)
                       for _ in range(3)),
        mesh=mesh,
        compiler_params=pltpu.CompilerParams(needs_layout_passes=False,
                                             use_tc_tiling_on_sc=False),
        scratch_types=[
            pltpu.VMEM((NP, 128), jnp.int32),
            pltpu.VMEM((NP, 128), jnp.int32),
            pltpu.VMEM((B, 8), jnp.int32),
            pltpu.VMEM((B, 8), jnp.int32),
            pltpu.VMEM((B, 8), jnp.int32),
            pltpu.VMEM((NV, 128), jnp.int32),
            pltpu.VMEM((3 * B, 8), jnp.float32),
            pltpu.VMEM((B,), jnp.float32),
            pltpu.VMEM((B,), jnp.float32),
            pltpu.VMEM((B,), jnp.float32),
            pltpu.VMEM((B,), jnp.float32),
            pltpu.VMEM((B,), jnp.float32),
            pltpu.VMEM((B,), jnp.float32),
            pltpu.VMEM((B,), jnp.float32),
            pltpu.VMEM((B,), jnp.float32),
            pltpu.VMEM((B,), jnp.float32),
            pltpu.SemaphoreType.DMA,
            pltpu.SemaphoreType.DMA,
        ],
    )
    def vertex_kernel(px_hbm, py_hbm, pz_hbm, nx_hbm, ny_hbm, nz_hbm,
                      prim_hbm, i0_hbm, i1_hbm, i2_hbm, comb_hbm,
                      ox_hbm, oy_hbm, oz_hbm,
                      prim_v, pidx_v, t0_v, t1_v, t2_v, vidx_v, vert_v,
                      hx_v, hy_v, hz_v, mx_v, my_v, mz_v,
                      ox_v, oy_v, oz_v, lsem, gsem):
        wid = lax.axis_index("s") * NC + lax.axis_index("c")
        base0 = wid * HW
        iota = lax.broadcasted_iota(jnp.int32, (L,), 0)

        col = [jnp.full((L,), c, jnp.int32) for c in range(8)]

        def chunk_body(ci, carry):
            base = base0 + ci * B

            descs = []
            for r in range(NP):
                descs.append(pltpu.async_copy(
                    prim_hbm.at[pl.ds(base + 128 * r, 128)],
                    prim_v.at[r], lsem))
            for src, dst in ((px_hbm, hx_v), (py_hbm, hy_v), (pz_hbm, hz_v),
                             (nx_hbm, mx_v), (ny_hbm, my_v), (nz_hbm, mz_v)):
                descs.append(pltpu.async_copy(src.at[pl.ds(base, B)], dst,
                                              lsem))
            for d in descs:
                d.wait()

            for r in range(NP):
                for k in range(128 // L):
                    p = prim_v[r, pl.ds(L * k, L)]
                    pidx_v[r, pl.ds(L * k, L)] = lax.shift_right_logical(p, 3)
            descs = []
            for r in range(NP):
                descs.append(pltpu.async_copy(
                    i0_hbm.at[pidx_v.at[r]],
                    t0_v.at[pl.ds(128 * r, 128)], gsem))
                descs.append(pltpu.async_copy(
                    i1_hbm.at[pidx_v.at[r]],
                    t1_v.at[pl.ds(128 * r, 128)], gsem))
                descs.append(pltpu.async_copy(
                    i2_hbm.at[pidx_v.at[r]],
                    t2_v.at[pl.ds(128 * r, 128)], gsem))
            for d in descs:
                d.wait()

            def repack(g, c2):
                h16 = g * L + iota
                prow = jnp.full((L,), 0, jnp.int32) + lax.div(g, 8)
                pcol = (lax.rem(g, 8) * L) + iota
                p16 = plsc.load_gather(prim_v, [prow, pcol])
                off = p16 & 7
                for s, t_v in ((0, t0_v), (1, t1_v), (2, t2_v)):
                    vals = plsc.load_gather(t_v, [h16, off])
                    p0 = s * B + g * L
                    row = jnp.full((L,), 0, jnp.int32) + lax.div(p0, 128)
                    c0 = lax.rem(p0, 128) + iota
                    plsc.store_scatter(vidx_v, [row, c0], vals)
                return c2
            lax.fori_loop(0, G, repack, 0, unroll=2)

            descs = []
            for r in range(NV):
                descs.append(pltpu.async_copy(
                    comb_hbm.at[vidx_v.at[r]],
                    vert_v.at[pl.ds(128 * r, 128)], gsem))
            for d in descs:
                d.wait()

            def compute(g, c2):
                h16 = g * L + iota
                hb = h16 + B
                hc = h16 + 2 * B
                ax = plsc.load_gather(vert_v, [h16, col[0]])
                ay = plsc.load_gather(vert_v, [h16, col[1]])
                az = plsc.load_gather(vert_v, [h16, col[2]])
                bx = plsc.load_gather(vert_v, [hb, col[0]])
                by = plsc.load_gather(vert_v, [hb, col[1]])
                bz = plsc.load_gather(vert_v, [hb, col[2]])
                cx = plsc.load_gather(vert_v, [hc, col[0]])
                cy = plsc.load_gather(vert_v, [hc, col[1]])
                cz = plsc.load_gather(vert_v, [hc, col[2]])
                px = plsc.load_gather(hx_v, [h16])
                py = plsc.load_gather(hy_v, [h16])
                pz = plsc.load_gather(hz_v, [h16])

                v0x, v0y, v0z = bx - ax, by - ay, bz - az
                v1x, v1y, v1z = cx - ax, cy - ay, cz - az
                v2x, v2y, v2z = px - ax, py - ay, pz - az
                d00 = v0x * v0x + v0y * v0y + v0z * v0z
                d01 = v0x * v1x + v0y * v1y + v0z * v1z
                d11 = v1x * v1x + v1y * v1y + v1z * v1z
                d20 = v2x * v0x + v2y * v0y + v2z * v0z
                d21 = v2x * v1x + v2y * v1y + v2z * v1z
                denom = d00 * d11 - d01 * d01
                denom = jnp.where(jnp.abs(denom) < 1e-12, 1e-12, denom)
                rden = 1.0 / denom
                v = (d11 * d20 - d01 * d21) * rden
                w = (d00 * d21 - d01 * d20) * rden
                u = 1.0 - v - w

                fax = plsc.load_gather(vert_v, [h16, col[3]])
                fay = plsc.load_gather(vert_v, [h16, col[4]])
                faz = plsc.load_gather(vert_v, [h16, col[5]])
                fbx = plsc.load_gather(vert_v, [hb, col[3]])
                fby = plsc.load_gather(vert_v, [hb, col[4]])
                fbz = plsc.load_gather(vert_v, [hb, col[5]])
                fcx = plsc.load_gather(vert_v, [hc, col[3]])
                fcy = plsc.load_gather(vert_v, [hc, col[4]])
                fcz = plsc.load_gather(vert_v, [hc, col[5]])
                nmx = u * fax + v * fbx + w * fcx
                nmy = u * fay + v * fby + w * fcy
                nmz = u * faz + v * fbz + w * fcz

                nx = plsc.load_gather(mx_v, [h16]) + (nmx * 0.5 - 0.25)
                ny = plsc.load_gather(my_v, [h16]) + (nmy * 0.5 - 0.25)
                nz = plsc.load_gather(mz_v, [h16]) + (nmz * 0.5 - 0.25)
                rn = _rsqrt(nx * nx + ny * ny + nz * nz)
                plsc.store_scatter(ox_v, [h16], nx * rn)
                plsc.store_scatter(oy_v, [h16], ny * rn)
                plsc.store_scatter(oz_v, [h16], nz * rn)
                return c2
            lax.fori_loop(0, G, compute, 0, unroll=2)

            pltpu.sync_copy(ox_v, ox_hbm.at[pl.ds(base, B)])
            pltpu.sync_copy(oy_v, oy_hbm.at[pl.ds(base, B)])
            pltpu.sync_copy(oz_v, oz_hbm.at[pl.ds(base, B)])
            return carry

        lax.fori_loop(0, NCHUNK, chunk_body, 0)

    return vertex_kernel


def kernel(hit_positions, hit_normals, hit_primIDs, vbo, ibo, features):
    H = hit_primIDs.shape[0]
    V = vbo.shape[0]
    T = ibo.shape[0]
    px, py, pz = (hit_positions[:, j] for j in range(3))
    nx, ny, nz = (hit_normals[:, j] for j in range(3))
    i0, i1, i2 = (ibo[:, j].reshape(T // 8, 8) for j in range(3))
    VP = 1 << 20
    pad = jnp.zeros((VP - V,), jnp.float32)
    cols = [jnp.concatenate([vbo[:, j], pad]) for j in range(3)]
    cols += [jnp.concatenate([features[:, j], pad]) for j in range(3)]
    comb = _make_interleave(VP, BV=1024)(*cols)
    k = _make_kernel(H, VP, T, B=512)
    ox, oy, oz = k(px, py, pz, nx, ny, nz, hit_primIDs, i0, i1, i2, comb)
    return jnp.stack([ox, oy, oz], axis=1)


# speedup vs baseline: 15.0849x; 1.1251x over previous
import functools

import jax
import jax.numpy as jnp
from jax import lax
from jax.experimental import pallas as pl
from jax.experimental.pallas import tpu as pltpu
from jax.experimental.pallas import tpu_sc as plsc

NC = 2
NS = 16
NW = NC * NS
L = 16


def _rsqrt(x):
    i = plsc.bitcast(x, jnp.int32)
    i = jnp.int32(0x5F3759DF) - lax.shift_right_logical(i, 1)
    y = plsc.bitcast(i, jnp.float32)
    xh = x * 0.5
    for _ in range(3):
        y = y * (1.5 - xh * y * y)
    return y


def _make_interleave(VP, BV):
    VW = VP // NW
    NCH = VW // BV
    G = BV // L
    assert VP % NW == 0 and VW % BV == 0 and BV % L == 0

    mesh = plsc.VectorSubcoreMesh(core_axis_name="c", subcore_axis_name="s",
                                  num_cores=NC, num_subcores=NS)

    @functools.partial(
        pl.kernel,
        out_type=jax.ShapeDtypeStruct((VP, 8), jnp.float32),
        mesh=mesh,
        compiler_params=pltpu.CompilerParams(needs_layout_passes=False,
                                             use_tc_tiling_on_sc=False),
        scratch_types=[
            pltpu.VMEM((BV,), jnp.float32),
            pltpu.VMEM((BV,), jnp.float32),
            pltpu.VMEM((BV,), jnp.float32),
            pltpu.VMEM((BV,), jnp.float32),
            pltpu.VMEM((BV,), jnp.float32),
            pltpu.VMEM((BV,), jnp.float32),
            pltpu.VMEM((BV, 8), jnp.float32),
            pltpu.SemaphoreType.DMA,
        ],
    )
    def interleave_kernel(c0, c1, c2, c3, c4, c5, comb_hbm,
                          v0, v1, v2, v3, v4, v5, cmb_v, lsem):
        wid = lax.axis_index("s") * NC + lax.axis_index("c")
        base0 = wid * VW
        iota = lax.broadcasted_iota(jnp.int32, (L,), 0)
        col = [jnp.full((L,), c, jnp.int32) for c in range(8)]
        zero = jnp.zeros((L,), jnp.float32)

        def zinit(g, c2_):
            n16 = g * L + iota
            plsc.store_scatter(cmb_v, [n16, col[6]], zero)
            plsc.store_scatter(cmb_v, [n16, col[7]], zero)
            return c2_
        lax.fori_loop(0, G, zinit, 0, unroll=2)

        def chunk_body(ci, carry):
            base = base0 + ci * BV
            descs = []
            for src, dst in ((c0, v0), (c1, v1), (c2, v2),
                             (c3, v3), (c4, v4), (c5, v5)):
                descs.append(pltpu.async_copy(src.at[pl.ds(base, BV)], dst,
                                              lsem))
            for d in descs:
                d.wait()

            def interleave(g, c2_):
                n16 = g * L + iota
                for c, v in ((0, v0), (1, v1), (2, v2),
                             (3, v3), (4, v4), (5, v5)):
                    vals = plsc.load_gather(v, [n16])
                    plsc.store_scatter(cmb_v, [n16, col[c]], vals)
                return c2_
            lax.fori_loop(0, G, interleave, 0, unroll=2)

            pltpu.sync_copy(cmb_v, comb_hbm.at[pl.ds(base, BV)])
            return carry

        lax.fori_loop(0, NCH, chunk_body, 0)

    return interleave_kernel


def _make_kernel(H, V, T, B):
    HW = H // NW
    NCHUNK = HW // B
    G = B // L
    NP = B // 128
    NV = 3 * B // 128
    assert H % NW == 0 and HW % B == 0 and B % 128 == 0

    mesh = plsc.VectorSubcoreMesh(core_axis_name="c", subcore_axis_name="s",
                                  num_cores=NC, num_subcores=NS)

    @functools.partial(
        pl.kernel,
        out_type=tuple(jax.ShapeDtypeStruct((H,), jnp.float32)
                       for _ in range(3)),
        mesh=mesh,
        compiler_params=pltpu.CompilerParams(needs_layout_passes=False,
                                             use_tc_tiling_on_sc=False),
        scratch_types=[
            pltpu.VMEM((2, NP, 128), jnp.int32),
            pltpu.VMEM((NP, 128), jnp.int32),
            pltpu.VMEM((B, 8), jnp.int32),
            pltpu.VMEM((B, 8), jnp.int32),
            pltpu.VMEM((B, 8), jnp.int32),
            pltpu.VMEM((NV, 128), jnp.int32),
            pltpu.VMEM((3 * B, 8), jnp.float32),
            pltpu.VMEM((2, B), jnp.float32),
            pltpu.VMEM((2, B), jnp.float32),
            pltpu.VMEM((2, B), jnp.float32),
            pltpu.VMEM((2, B), jnp.float32),
            pltpu.VMEM((2, B), jnp.float32),
            pltpu.VMEM((2, B), jnp.float32),
            pltpu.VMEM((B,), jnp.float32),
            pltpu.VMEM((B,), jnp.float32),
            pltpu.VMEM((B,), jnp.float32),
            pltpu.SemaphoreType.DMA,
            pltpu.SemaphoreType.DMA,
        ],
    )
    def vertex_kernel(px_hbm, py_hbm, pz_hbm, nx_hbm, ny_hbm, nz_hbm,
                      prim_hbm, i0_hbm, i1_hbm, i2_hbm, comb_hbm,
                      ox_hbm, oy_hbm, oz_hbm,
                      prim_v, pidx_v, t0_v, t1_v, t2_v, vidx_v, vert_v,
                      hx_v, hy_v, hz_v, mx_v, my_v, mz_v,
                      ox_v, oy_v, oz_v, lsem, gsem):
        wid = lax.axis_index("s") * NC + lax.axis_index("c")
        base0 = wid * HW
        iota = lax.broadcasted_iota(jnp.int32, (L,), 0)

        col = [jnp.full((L,), c, jnp.int32) for c in range(8)]

        def issue_lin(base, s):
            for r in range(NP):
                pltpu.async_copy(prim_hbm.at[pl.ds(base + 128 * r, 128)],
                                 prim_v.at[s].at[r], lsem)
            for src, dst in ((px_hbm, hx_v), (py_hbm, hy_v), (pz_hbm, hz_v),
                             (nx_hbm, mx_v), (ny_hbm, my_v), (nz_hbm, mz_v)):
                pltpu.async_copy(src.at[pl.ds(base, B)], dst.at[s], lsem)

        issue_lin(base0, 0)

        def chunk_body(ci, carry):
            base = base0 + ci * B
            sl = lax.rem(ci, 2)

            for r in range(NP):
                pltpu.make_async_copy(
                    prim_hbm.at[pl.ds(base + 128 * r, 128)],
                    prim_v.at[sl].at[r], lsem).wait()
            for src, dst in ((px_hbm, hx_v), (py_hbm, hy_v), (pz_hbm, hz_v),
                             (nx_hbm, mx_v), (ny_hbm, my_v), (nz_hbm, mz_v)):
                pltpu.make_async_copy(src.at[pl.ds(base, B)], dst.at[sl],
                                      lsem).wait()

            @pl.when(ci + 1 < NCHUNK)
            def _prefetch():
                issue_lin(base + B, 1 - sl)

            for r in range(NP):
                for k in range(128 // L):
                    p = prim_v[sl, r, pl.ds(L * k, L)]
                    pidx_v[r, pl.ds(L * k, L)] = lax.shift_right_logical(p, 3)
            descs = []
            for r in range(NP):
                descs.append(pltpu.async_copy(
                    i0_hbm.at[pidx_v.at[r]],
                    t0_v.at[pl.ds(128 * r, 128)], gsem))
                descs.append(pltpu.async_copy(
                    i1_hbm.at[pidx_v.at[r]],
                    t1_v.at[pl.ds(128 * r, 128)], gsem))
                descs.append(pltpu.async_copy(
                    i2_hbm.at[pidx_v.at[r]],
                    t2_v.at[pl.ds(128 * r, 128)], gsem))
            for d in descs:
                d.wait()

            def repack(g, c2):
                h16 = g * L + iota
                prow = jnp.full((L,), 0, jnp.int32) + lax.div(g, 8)
                pcol = (lax.rem(g, 8) * L) + iota
                p16 = plsc.load_gather(prim_v.at[sl], [prow, pcol])
                off = p16 & 7
                for s, t_v in ((0, t0_v), (1, t1_v), (2, t2_v)):
                    vals = plsc.load_gather(t_v, [h16, off])
                    p0 = s * B + g * L
                    row = jnp.full((L,), 0, jnp.int32) + lax.div(p0, 128)
                    c0 = lax.rem(p0, 128) + iota
                    plsc.store_scatter(vidx_v, [row, c0], vals)
                return c2
            lax.fori_loop(0, G, repack, 0, unroll=2)

            descs = []
            for r in range(NV):
                descs.append(pltpu.async_copy(
                    comb_hbm.at[vidx_v.at[r]],
                    vert_v.at[pl.ds(128 * r, 128)], gsem))
            for d in descs:
                d.wait()

            def compute(g, c2):
                h16 = g * L + iota
                hb = h16 + B
                hc = h16 + 2 * B
                ax = plsc.load_gather(vert_v, [h16, col[0]])
                ay = plsc.load_gather(vert_v, [h16, col[1]])
                az = plsc.load_gather(vert_v, [h16, col[2]])
                bx = plsc.load_gather(vert_v, [hb, col[0]])
                by = plsc.load_gather(vert_v, [hb, col[1]])
                bz = plsc.load_gather(vert_v, [hb, col[2]])
                cx = plsc.load_gather(vert_v, [hc, col[0]])
                cy = plsc.load_gather(vert_v, [hc, col[1]])
                cz = plsc.load_gather(vert_v, [hc, col[2]])
                px = plsc.load_gather(hx_v.at[sl], [h16])
                py = plsc.load_gather(hy_v.at[sl], [h16])
                pz = plsc.load_gather(hz_v.at[sl], [h16])

                v0x, v0y, v0z = bx - ax, by - ay, bz - az
                v1x, v1y, v1z = cx - ax, cy - ay, cz - az
                v2x, v2y, v2z = px - ax, py - ay, pz - az
                d00 = v0x * v0x + v0y * v0y + v0z * v0z
                d01 = v0x * v1x + v0y * v1y + v0z * v1z
                d11 = v1x * v1x + v1y * v1y + v1z * v1z
                d20 = v2x * v0x + v2y * v0y + v2z * v0z
                d21 = v2x * v1x + v2y * v1y + v2z * v1z
                denom = d00 * d11 - d01 * d01
                denom = jnp.where(jnp.abs(denom) < 1e-12, 1e-12, denom)
                rden = 1.0 / denom
                v = (d11 * d20 - d01 * d21) * rden
                w = (d00 * d21 - d01 * d20) * rden
                u = 1.0 - v - w

                fax = plsc.load_gather(vert_v, [h16, col[3]])
                fay = plsc.load_gather(vert_v, [h16, col[4]])
                faz = plsc.load_gather(vert_v, [h16, col[5]])
                fbx = plsc.load_gather(vert_v, [hb, col[3]])
                fby = plsc.load_gather(vert_v, [hb, col[4]])
                fbz = plsc.load_gather(vert_v, [hb, col[5]])
                fcx = plsc.load_gather(vert_v, [hc, col[3]])
                fcy = plsc.load_gather(vert_v, [hc, col[4]])
                fcz = plsc.load_gather(vert_v, [hc, col[5]])
                nmx = u * fax + v * fbx + w * fcx
                nmy = u * fay + v * fby + w * fcy
                nmz = u * faz + v * fbz + w * fcz

                nx = plsc.load_gather(mx_v.at[sl], [h16]) + (nmx * 0.5 - 0.25)
                ny = plsc.load_gather(my_v.at[sl], [h16]) + (nmy * 0.5 - 0.25)
                nz = plsc.load_gather(mz_v.at[sl], [h16]) + (nmz * 0.5 - 0.25)
                rn = _rsqrt(nx * nx + ny * ny + nz * nz)
                plsc.store_scatter(ox_v, [h16], nx * rn)
                plsc.store_scatter(oy_v, [h16], ny * rn)
                plsc.store_scatter(oz_v, [h16], nz * rn)
                return c2
            lax.fori_loop(0, G, compute, 0, unroll=2)

            pltpu.sync_copy(ox_v, ox_hbm.at[pl.ds(base, B)])
            pltpu.sync_copy(oy_v, oy_hbm.at[pl.ds(base, B)])
            pltpu.sync_copy(oz_v, oz_hbm.at[pl.ds(base, B)])
            return carry

        lax.fori_loop(0, NCHUNK, chunk_body, 0)

    return vertex_kernel


def kernel(hit_positions, hit_normals, hit_primIDs, vbo, ibo, features):
    H = hit_primIDs.shape[0]
    V = vbo.shape[0]
    T = ibo.shape[0]
    px, py, pz = (hit_positions[:, j] for j in range(3))
    nx, ny, nz = (hit_normals[:, j] for j in range(3))
    i0, i1, i2 = (ibo[:, j].reshape(T // 8, 8) for j in range(3))
    VP = 1 << 20
    pad = jnp.zeros((VP - V,), jnp.float32)
    cols = [jnp.concatenate([vbo[:, j], pad]) for j in range(3)]
    cols += [jnp.concatenate([features[:, j], pad]) for j in range(3)]
    comb = _make_interleave(VP, BV=4096)(*cols)
    k = _make_kernel(H, VP, T, B=1024)
    ox, oy, oz = k(px, py, pz, nx, ny, nz, hit_primIDs, i0, i1, i2, comb)
    return jnp.stack([ox, oy, oz], axis=1)


# speedup vs baseline: 18.6491x; 1.2363x over previous
import functools

import jax
import jax.numpy as jnp
from jax import lax
from jax.experimental import pallas as pl
from jax.experimental.pallas import tpu as pltpu
from jax.experimental.pallas import tpu_sc as plsc

NC = 2
NS = 16
NW = NC * NS
L = 16


def _rsqrt(x):
    i = plsc.bitcast(x, jnp.int32)
    i = jnp.int32(0x5F3759DF) - lax.shift_right_logical(i, 1)
    y = plsc.bitcast(i, jnp.float32)
    xh = x * 0.5
    for _ in range(3):
        y = y * (1.5 - xh * y * y)
    return y


def _make_interleave(VP, BV):
    VW = VP // NW
    NCH = VW // BV
    G = BV // L
    assert VP % NW == 0 and VW % BV == 0 and BV % L == 0

    mesh = plsc.VectorSubcoreMesh(core_axis_name="c", subcore_axis_name="s",
                                  num_cores=NC, num_subcores=NS)

    @functools.partial(
        pl.kernel,
        out_type=jax.ShapeDtypeStruct((VP, 8), jnp.float32),
        mesh=mesh,
        compiler_params=pltpu.CompilerParams(needs_layout_passes=False,
                                             use_tc_tiling_on_sc=False),
        scratch_types=[
            pltpu.VMEM((BV,), jnp.float32),
            pltpu.VMEM((BV,), jnp.float32),
            pltpu.VMEM((BV,), jnp.float32),
            pltpu.VMEM((BV,), jnp.float32),
            pltpu.VMEM((BV,), jnp.float32),
            pltpu.VMEM((BV,), jnp.float32),
            pltpu.VMEM((BV, 8), jnp.float32),
            pltpu.SemaphoreType.DMA,
        ],
    )
    def interleave_kernel(c0, c1, c2, c3, c4, c5, comb_hbm,
                          v0, v1, v2, v3, v4, v5, cmb_v, lsem):
        wid = lax.axis_index("s") * NC + lax.axis_index("c")
        base0 = wid * VW
        iota = lax.broadcasted_iota(jnp.int32, (L,), 0)
        col = [jnp.full((L,), c, jnp.int32) for c in range(8)]
        zero = jnp.zeros((L,), jnp.float32)

        def zinit(g, c2_):
            n16 = g * L + iota
            plsc.store_scatter(cmb_v, [n16, col[6]], zero)
            plsc.store_scatter(cmb_v, [n16, col[7]], zero)
            return c2_
        lax.fori_loop(0, G, zinit, 0, unroll=2)

        def chunk_body(ci, carry):
            base = base0 + ci * BV
            descs = []
            for src, dst in ((c0, v0), (c1, v1), (c2, v2),
                             (c3, v3), (c4, v4), (c5, v5)):
                descs.append(pltpu.async_copy(src.at[pl.ds(base, BV)], dst,
                                              lsem))
            for d in descs:
                d.wait()

            def interleave(g, c2_):
                n16 = g * L + iota
                for c, v in ((0, v0), (1, v1), (2, v2),
                             (3, v3), (4, v4), (5, v5)):
                    vals = plsc.load_gather(v, [n16])
                    plsc.store_scatter(cmb_v, [n16, col[c]], vals)
                return c2_
            lax.fori_loop(0, G, interleave, 0, unroll=2)

            pltpu.sync_copy(cmb_v, comb_hbm.at[pl.ds(base, BV)])
            return carry

        lax.fori_loop(0, NCH, chunk_body, 0)

    return interleave_kernel


def _make_kernel(H, V, T, B):
    HW = H // NW
    NCHUNK = HW // B
    G = B // L
    NP = B // 128
    NV = 3 * B // 128
    assert H % NW == 0 and HW % B == 0 and B % 128 == 0

    mesh = plsc.VectorSubcoreMesh(core_axis_name="c", subcore_axis_name="s",
                                  num_cores=NC, num_subcores=NS)

    @functools.partial(
        pl.kernel,
        out_type=tuple(jax.ShapeDtypeStruct((H,), jnp.float32)
                       for _ in range(3)),
        mesh=mesh,
        compiler_params=pltpu.CompilerParams(needs_layout_passes=False,
                                             use_tc_tiling_on_sc=False),
        scratch_types=[
            pltpu.VMEM((2, NP, 128), jnp.int32),
            pltpu.VMEM((NP, 128), jnp.int32),
            pltpu.VMEM((B, 8), jnp.int32),
            pltpu.VMEM((B, 8), jnp.int32),
            pltpu.VMEM((B, 8), jnp.int32),
            pltpu.VMEM((NV, 128), jnp.int32),
            pltpu.VMEM((3 * B, 8), jnp.float32),
            pltpu.VMEM((2, B), jnp.float32),
            pltpu.VMEM((2, B), jnp.float32),
            pltpu.VMEM((2, B), jnp.float32),
            pltpu.VMEM((2, B), jnp.float32),
            pltpu.VMEM((2, B), jnp.float32),
            pltpu.VMEM((2, B), jnp.float32),
            pltpu.VMEM((B,), jnp.float32),
            pltpu.VMEM((B,), jnp.float32),
            pltpu.VMEM((B,), jnp.float32),
            pltpu.SemaphoreType.DMA,
            pltpu.SemaphoreType.DMA,
            pltpu.SemaphoreType.DMA,
            pltpu.SemaphoreType.DMA,
            pltpu.SemaphoreType.DMA,
            pltpu.SemaphoreType.DMA,
            pltpu.SemaphoreType.DMA,
            pltpu.SemaphoreType.DMA,
            pltpu.SemaphoreType.DMA,
        ],
    )
    def vertex_kernel(px_hbm, py_hbm, pz_hbm, nx_hbm, ny_hbm, nz_hbm,
                      prim_hbm, i0_hbm, i1_hbm, i2_hbm, comb_hbm,
                      ox_hbm, oy_hbm, oz_hbm,
                      prim_v, pidx_v, t0_v, t1_v, t2_v, vidx_v, vert_v,
                      hx_v, hy_v, hz_v, mx_v, my_v, mz_v,
                      ox_v, oy_v, oz_v, lsem,
                      g1s0, g1s1, g1s2, g1s3, g2s0, g2s1, g2s2, g2s3):
        wid = lax.axis_index("s") * NC + lax.axis_index("c")
        base0 = wid * HW
        iota = lax.broadcasted_iota(jnp.int32, (L,), 0)

        col = [jnp.full((L,), c, jnp.int32) for c in range(8)]

        def issue_lin(base, s):
            for r in range(NP):
                pltpu.async_copy(prim_hbm.at[pl.ds(base + 128 * r, 128)],
                                 prim_v.at[s].at[r], lsem)
            for src, dst in ((px_hbm, hx_v), (py_hbm, hy_v), (pz_hbm, hz_v),
                             (nx_hbm, mx_v), (ny_hbm, my_v), (nz_hbm, mz_v)):
                pltpu.async_copy(src.at[pl.ds(base, B)], dst.at[s], lsem)

        issue_lin(base0, 0)

        def chunk_body(ci, carry):
            base = base0 + ci * B
            sl = lax.rem(ci, 2)

            for r in range(NP):
                pltpu.make_async_copy(
                    prim_hbm.at[pl.ds(base + 128 * r, 128)],
                    prim_v.at[sl].at[r], lsem).wait()
            for src, dst in ((px_hbm, hx_v), (py_hbm, hy_v), (pz_hbm, hz_v),
                             (nx_hbm, mx_v), (ny_hbm, my_v), (nz_hbm, mz_v)):
                pltpu.make_async_copy(src.at[pl.ds(base, B)], dst.at[sl],
                                      lsem).wait()

            @pl.when(ci + 1 < NCHUNK)
            def _prefetch():
                issue_lin(base + B, 1 - sl)

            for r in range(NP):
                for k in range(128 // L):
                    p = prim_v[sl, r, pl.ds(L * k, L)]
                    pidx_v[r, pl.ds(L * k, L)] = lax.shift_right_logical(p, 3)
            g1sems = (g1s0, g1s1, g1s2, g1s3)
            g2sems = (g2s0, g2s1, g2s2, g2s3)
            NQ = 4
            RQ = NP // NQ
            g1descs = [[], [], [], []]
            for q in range(NQ):
                for r in range(RQ * q, RQ * (q + 1)):
                    for tbl, t_v in ((i0_hbm, t0_v), (i1_hbm, t1_v),
                                     (i2_hbm, t2_v)):
                        g1descs[q].append(pltpu.async_copy(
                            tbl.at[pidx_v.at[r]],
                            t_v.at[pl.ds(128 * r, 128)], g1sems[q]))

            def repack(g, c2):
                h16 = g * L + iota
                prow = jnp.full((L,), 0, jnp.int32) + lax.div(g, 8)
                pcol = (lax.rem(g, 8) * L) + iota
                p16 = plsc.load_gather(prim_v.at[sl], [prow, pcol])
                off = p16 & 7
                for s, t_v in ((0, t0_v), (1, t1_v), (2, t2_v)):
                    vals = plsc.load_gather(t_v, [h16, off])
                    p0 = s * B + g * L
                    row = jnp.full((L,), 0, jnp.int32) + lax.div(p0, 128)
                    c0 = lax.rem(p0, 128) + iota
                    plsc.store_scatter(vidx_v, [row, c0], vals)
                return c2
            GQ = G // NQ
            VQ = NV // 3 // NQ
            g2descs = [[], [], [], []]
            for q in range(NQ):
                for d in g1descs[q]:
                    d.wait()
                lax.fori_loop(GQ * q, GQ * (q + 1), repack, 0, unroll=2)
                for s in range(3):
                    for rr in range(s * (NV // 3) + VQ * q,
                                    s * (NV // 3) + VQ * (q + 1)):
                        g2descs[q].append(pltpu.async_copy(
                            comb_hbm.at[vidx_v.at[rr]],
                            vert_v.at[pl.ds(128 * rr, 128)], g2sems[q]))

            def compute(g, c2):
                h16 = g * L + iota
                hb = h16 + B
                hc = h16 + 2 * B
                ax = plsc.load_gather(vert_v, [h16, col[0]])
                ay = plsc.load_gather(vert_v, [h16, col[1]])
                az = plsc.load_gather(vert_v, [h16, col[2]])
                bx = plsc.load_gather(vert_v, [hb, col[0]])
                by = plsc.load_gather(vert_v, [hb, col[1]])
                bz = plsc.load_gather(vert_v, [hb, col[2]])
                cx = plsc.load_gather(vert_v, [hc, col[0]])
                cy = plsc.load_gather(vert_v, [hc, col[1]])
                cz = plsc.load_gather(vert_v, [hc, col[2]])
                px = plsc.load_gather(hx_v.at[sl], [h16])
                py = plsc.load_gather(hy_v.at[sl], [h16])
                pz = plsc.load_gather(hz_v.at[sl], [h16])

                v0x, v0y, v0z = bx - ax, by - ay, bz - az
                v1x, v1y, v1z = cx - ax, cy - ay, cz - az
                v2x, v2y, v2z = px - ax, py - ay, pz - az
                d00 = v0x * v0x + v0y * v0y + v0z * v0z
                d01 = v0x * v1x + v0y * v1y + v0z * v1z
                d11 = v1x * v1x + v1y * v1y + v1z * v1z
                d20 = v2x * v0x + v2y * v0y + v2z * v0z
                d21 = v2x * v1x + v2y * v1y + v2z * v1z
                denom = d00 * d11 - d01 * d01
                denom = jnp.where(jnp.abs(denom) < 1e-12, 1e-12, denom)
                rden = 1.0 / denom
                v = (d11 * d20 - d01 * d21) * rden
                w = (d00 * d21 - d01 * d20) * rden
                u = 1.0 - v - w

                fax = plsc.load_gather(vert_v, [h16, col[3]])
                fay = plsc.load_gather(vert_v, [h16, col[4]])
                faz = plsc.load_gather(vert_v, [h16, col[5]])
                fbx = plsc.load_gather(vert_v, [hb, col[3]])
                fby = plsc.load_gather(vert_v, [hb, col[4]])
                fbz = plsc.load_gather(vert_v, [hb, col[5]])
                fcx = plsc.load_gather(vert_v, [hc, col[3]])
                fcy = plsc.load_gather(vert_v, [hc, col[4]])
                fcz = plsc.load_gather(vert_v, [hc, col[5]])
                nmx = u * fax + v * fbx + w * fcx
                nmy = u * fay + v * fby + w * fcy
                nmz = u * faz + v * fbz + w * fcz

                nx = plsc.load_gather(mx_v.at[sl], [h16]) + (nmx * 0.5 - 0.25)
                ny = plsc.load_gather(my_v.at[sl], [h16]) + (nmy * 0.5 - 0.25)
                nz = plsc.load_gather(mz_v.at[sl], [h16]) + (nmz * 0.5 - 0.25)
                rn = _rsqrt(nx * nx + ny * ny + nz * nz)
                plsc.store_scatter(ox_v, [h16], nx * rn)
                plsc.store_scatter(oy_v, [h16], ny * rn)
                plsc.store_scatter(oz_v, [h16], nz * rn)
                return c2
            for q in range(NQ):
                for d in g2descs[q]:
                    d.wait()
                lax.fori_loop(GQ * q, GQ * (q + 1), compute, 0, unroll=2)

            pltpu.sync_copy(ox_v, ox_hbm.at[pl.ds(base, B)])
            pltpu.sync_copy(oy_v, oy_hbm.at[pl.ds(base, B)])
            pltpu.sync_copy(oz_v, oz_hbm.at[pl.ds(base, B)])
            return carry

        lax.fori_loop(0, NCHUNK, chunk_body, 0)

    return vertex_kernel


def kernel(hit_positions, hit_normals, hit_primIDs, vbo, ibo, features):
    H = hit_primIDs.shape[0]
    V = vbo.shape[0]
    T = ibo.shape[0]
    px, py, pz = (hit_positions[:, j] for j in range(3))
    nx, ny, nz = (hit_normals[:, j] for j in range(3))
    i0, i1, i2 = (ibo[:, j].reshape(T // 8, 8) for j in range(3))
    VP = 1 << 20
    pad = jnp.zeros((VP - V,), jnp.float32)
    cols = [jnp.concatenate([vbo[:, j], pad]) for j in range(3)]
    cols += [jnp.concatenate([features[:, j], pad]) for j in range(3)]
    comb = _make_interleave(VP, BV=4096)(*cols)
    k = _make_kernel(H, VP, T, B=1024)
    ox, oy, oz = k(px, py, pz, nx, ny, nz, hit_primIDs, i0, i1, i2, comb)
    return jnp.stack([ox, oy, oz], axis=1)


# speedup vs baseline: 19.5089x; 1.0461x over previous
import functools

import jax
import jax.numpy as jnp
from jax import lax
from jax.experimental import pallas as pl
from jax.experimental.pallas import tpu as pltpu
from jax.experimental.pallas import tpu_sc as plsc

NC = 2
NS = 16
NW = NC * NS
L = 16


def _rsqrt(x):
    i = plsc.bitcast(x, jnp.int32)
    i = jnp.int32(0x5F3759DF) - lax.shift_right_logical(i, 1)
    y = plsc.bitcast(i, jnp.float32)
    xh = x * 0.5
    for _ in range(3):
        y = y * (1.5 - xh * y * y)
    return y


def _make_interleave(VP, BV):
    VW = VP // NW
    NCH = VW // BV
    G = BV // L
    assert VP % NW == 0 and VW % BV == 0 and BV % L == 0

    mesh = plsc.VectorSubcoreMesh(core_axis_name="c", subcore_axis_name="s",
                                  num_cores=NC, num_subcores=NS)

    @functools.partial(
        pl.kernel,
        out_type=jax.ShapeDtypeStruct((VP, 8), jnp.float32),
        mesh=mesh,
        compiler_params=pltpu.CompilerParams(needs_layout_passes=False,
                                             use_tc_tiling_on_sc=False),
        scratch_types=[
            pltpu.VMEM((BV,), jnp.float32),
            pltpu.VMEM((BV,), jnp.float32),
            pltpu.VMEM((BV,), jnp.float32),
            pltpu.VMEM((BV,), jnp.float32),
            pltpu.VMEM((BV,), jnp.float32),
            pltpu.VMEM((BV,), jnp.float32),
            pltpu.VMEM((BV, 8), jnp.float32),
            pltpu.SemaphoreType.DMA,
        ],
    )
    def interleave_kernel(c0, c1, c2, c3, c4, c5, comb_hbm,
                          v0, v1, v2, v3, v4, v5, cmb_v, lsem):
        wid = lax.axis_index("s") * NC + lax.axis_index("c")
        base0 = wid * VW
        iota = lax.broadcasted_iota(jnp.int32, (L,), 0)
        col = [jnp.full((L,), c, jnp.int32) for c in range(8)]
        zero = jnp.zeros((L,), jnp.float32)

        def zinit(g, c2_):
            n16 = g * L + iota
            plsc.store_scatter(cmb_v, [n16, col[6]], zero)
            plsc.store_scatter(cmb_v, [n16, col[7]], zero)
            return c2_
        lax.fori_loop(0, G, zinit, 0, unroll=2)

        def chunk_body(ci, carry):
            base = base0 + ci * BV
            descs = []
            for src, dst in ((c0, v0), (c1, v1), (c2, v2),
                             (c3, v3), (c4, v4), (c5, v5)):
                descs.append(pltpu.async_copy(src.at[pl.ds(base, BV)], dst,
                                              lsem))
            for d in descs:
                d.wait()

            def interleave(g, c2_):
                n16 = g * L + iota
                for c, v in ((0, v0), (1, v1), (2, v2),
                             (3, v3), (4, v4), (5, v5)):
                    vals = plsc.load_gather(v, [n16])
                    plsc.store_scatter(cmb_v, [n16, col[c]], vals)
                return c2_
            lax.fori_loop(0, G, interleave, 0, unroll=2)

            pltpu.sync_copy(cmb_v, comb_hbm.at[pl.ds(base, BV)])
            return carry

        lax.fori_loop(0, NCH, chunk_body, 0)

    return interleave_kernel


def _make_kernel(H, V, T, B):
    HW = H // NW
    NCHUNK = HW // B
    G = B // L
    NP = B // 128
    NV = 3 * B // 128
    assert H % NW == 0 and HW % B == 0 and B % 128 == 0

    mesh = plsc.VectorSubcoreMesh(core_axis_name="c", subcore_axis_name="s",
                                  num_cores=NC, num_subcores=NS)

    @functools.partial(
        pl.kernel,
        out_type=tuple(jax.ShapeDtypeStruct((H,), jnp.float32)
                       for _ in range(3)),
        mesh=mesh,
        compiler_params=pltpu.CompilerParams(needs_layout_passes=False,
                                             use_tc_tiling_on_sc=False),
        scratch_types=[
            pltpu.VMEM((2, NP, 128), jnp.int32),
            pltpu.VMEM((NP, 128), jnp.int32),
            pltpu.VMEM((B, 8), jnp.int32),
            pltpu.VMEM((B, 8), jnp.int32),
            pltpu.VMEM((B, 8), jnp.int32),
            pltpu.VMEM((NV, 128), jnp.int32),
            pltpu.VMEM((3 * B, 8), jnp.float32),
            pltpu.VMEM((2, B), jnp.float32),
            pltpu.VMEM((2, B), jnp.float32),
            pltpu.VMEM((2, B), jnp.float32),
            pltpu.VMEM((2, B), jnp.float32),
            pltpu.VMEM((2, B), jnp.float32),
            pltpu.VMEM((2, B), jnp.float32),
            pltpu.VMEM((B,), jnp.float32),
            pltpu.VMEM((B,), jnp.float32),
            pltpu.VMEM((B,), jnp.float32),
            pltpu.SemaphoreType.DMA,
            pltpu.SemaphoreType.DMA,
            pltpu.SemaphoreType.DMA,
            pltpu.SemaphoreType.DMA,
            pltpu.SemaphoreType.DMA,
            pltpu.SemaphoreType.DMA,
            pltpu.SemaphoreType.DMA,
            pltpu.SemaphoreType.DMA,
            pltpu.SemaphoreType.DMA,
            pltpu.SemaphoreType.DMA,
            pltpu.SemaphoreType.DMA,
            pltpu.SemaphoreType.DMA,
            pltpu.SemaphoreType.DMA,
            pltpu.SemaphoreType.DMA,
            pltpu.SemaphoreType.DMA,
            pltpu.SemaphoreType.DMA,
            pltpu.SemaphoreType.DMA,
        ],
    )
    def vertex_kernel(px_hbm, py_hbm, pz_hbm, nx_hbm, ny_hbm, nz_hbm,
                      prim_hbm, i0_hbm, i1_hbm, i2_hbm, comb_hbm,
                      ox_hbm, oy_hbm, oz_hbm,
                      prim_v, pidx_v, t0_v, t1_v, t2_v, vidx_v, vert_v,
                      hx_v, hy_v, hz_v, mx_v, my_v, mz_v,
                      ox_v, oy_v, oz_v, lsem,
                      g1s0, g1s1, g1s2, g1s3, g1s4, g1s5, g1s6, g1s7,
                      g2s0, g2s1, g2s2, g2s3, g2s4, g2s5, g2s6, g2s7):
        wid = lax.axis_index("s") * NC + lax.axis_index("c")
        base0 = wid * HW
        iota = lax.broadcasted_iota(jnp.int32, (L,), 0)

        col = [jnp.full((L,), c, jnp.int32) for c in range(8)]

        def issue_lin(base, s):
            for r in range(NP):
                pltpu.async_copy(prim_hbm.at[pl.ds(base + 128 * r, 128)],
                                 prim_v.at[s].at[r], lsem)
            for src, dst in ((px_hbm, hx_v), (py_hbm, hy_v), (pz_hbm, hz_v),
                             (nx_hbm, mx_v), (ny_hbm, my_v), (nz_hbm, mz_v)):
                pltpu.async_copy(src.at[pl.ds(base, B)], dst.at[s], lsem)

        issue_lin(base0, 0)

        def chunk_body(ci, carry):
            base = base0 + ci * B
            sl = lax.rem(ci, 2)

            for r in range(NP):
                pltpu.make_async_copy(
                    prim_hbm.at[pl.ds(base + 128 * r, 128)],
                    prim_v.at[sl].at[r], lsem).wait()
            for src, dst in ((px_hbm, hx_v), (py_hbm, hy_v), (pz_hbm, hz_v),
                             (nx_hbm, mx_v), (ny_hbm, my_v), (nz_hbm, mz_v)):
                pltpu.make_async_copy(src.at[pl.ds(base, B)], dst.at[sl],
                                      lsem).wait()

            @pl.when(ci + 1 < NCHUNK)
            def _prefetch():
                issue_lin(base + B, 1 - sl)

            for r in range(NP):
                for k in range(128 // L):
                    p = prim_v[sl, r, pl.ds(L * k, L)]
                    pidx_v[r, pl.ds(L * k, L)] = lax.shift_right_logical(p, 3)
            g1sems = (g1s0, g1s1, g1s2, g1s3, g1s4, g1s5, g1s6, g1s7)
            g2sems = (g2s0, g2s1, g2s2, g2s3, g2s4, g2s5, g2s6, g2s7)
            NQ = 8
            RQ = NP // NQ
            g1descs = [[] for _ in range(NQ)]
            for q in range(NQ):
                for r in range(RQ * q, RQ * (q + 1)):
                    for tbl, t_v in ((i0_hbm, t0_v), (i1_hbm, t1_v),
                                     (i2_hbm, t2_v)):
                        g1descs[q].append(pltpu.async_copy(
                            tbl.at[pidx_v.at[r]],
                            t_v.at[pl.ds(128 * r, 128)], g1sems[q]))

            def repack(g, c2):
                h16 = g * L + iota
                prow = jnp.full((L,), 0, jnp.int32) + lax.div(g, 8)
                pcol = (lax.rem(g, 8) * L) + iota
                p16 = plsc.load_gather(prim_v.at[sl], [prow, pcol])
                off = p16 & 7
                for s, t_v in ((0, t0_v), (1, t1_v), (2, t2_v)):
                    vals = plsc.load_gather(t_v, [h16, off])
                    p0 = s * B + g * L
                    row = jnp.full((L,), 0, jnp.int32) + lax.div(p0, 128)
                    c0 = lax.rem(p0, 128) + iota
                    plsc.store_scatter(vidx_v, [row, c0], vals)
                return c2
            GQ = G // NQ
            VQ = NV // 3 // NQ
            g2descs = [[] for _ in range(NQ)]
            for q in range(NQ):
                for d in g1descs[q]:
                    d.wait()
                lax.fori_loop(GQ * q, GQ * (q + 1), repack, 0, unroll=2)
                for s in range(3):
                    for rr in range(s * (NV // 3) + VQ * q,
                                    s * (NV // 3) + VQ * (q + 1)):
                        g2descs[q].append(pltpu.async_copy(
                            comb_hbm.at[vidx_v.at[rr]],
                            vert_v.at[pl.ds(128 * rr, 128)], g2sems[q]))

            def compute(g, c2):
                h16 = g * L + iota
                hb = h16 + B
                hc = h16 + 2 * B
                ax = plsc.load_gather(vert_v, [h16, col[0]])
                ay = plsc.load_gather(vert_v, [h16, col[1]])
                az = plsc.load_gather(vert_v, [h16, col[2]])
                bx = plsc.load_gather(vert_v, [hb, col[0]])
                by = plsc.load_gather(vert_v, [hb, col[1]])
                bz = plsc.load_gather(vert_v, [hb, col[2]])
                cx = plsc.load_gather(vert_v, [hc, col[0]])
                cy = plsc.load_gather(vert_v, [hc, col[1]])
                cz = plsc.load_gather(vert_v, [hc, col[2]])
                px = plsc.load_gather(hx_v.at[sl], [h16])
                py = plsc.load_gather(hy_v.at[sl], [h16])
                pz = plsc.load_gather(hz_v.at[sl], [h16])

                v0x, v0y, v0z = bx - ax, by - ay, bz - az
                v1x, v1y, v1z = cx - ax, cy - ay, cz - az
                v2x, v2y, v2z = px - ax, py - ay, pz - az
                d00 = v0x * v0x + v0y * v0y + v0z * v0z
                d01 = v0x * v1x + v0y * v1y + v0z * v1z
                d11 = v1x * v1x + v1y * v1y + v1z * v1z
                d20 = v2x * v0x + v2y * v0y + v2z * v0z
                d21 = v2x * v1x + v2y * v1y + v2z * v1z
                denom = d00 * d11 - d01 * d01
                denom = jnp.where(jnp.abs(denom) < 1e-12, 1e-12, denom)
                rden = 1.0 / denom
                v = (d11 * d20 - d01 * d21) * rden
                w = (d00 * d21 - d01 * d20) * rden
                u = 1.0 - v - w

                fax = plsc.load_gather(vert_v, [h16, col[3]])
                fay = plsc.load_gather(vert_v, [h16, col[4]])
                faz = plsc.load_gather(vert_v, [h16, col[5]])
                fbx = plsc.load_gather(vert_v, [hb, col[3]])
                fby = plsc.load_gather(vert_v, [hb, col[4]])
                fbz = plsc.load_gather(vert_v, [hb, col[5]])
                fcx = plsc.load_gather(vert_v, [hc, col[3]])
                fcy = plsc.load_gather(vert_v, [hc, col[4]])
                fcz = plsc.load_gather(vert_v, [hc, col[5]])
                nmx = u * fax + v * fbx + w * fcx
                nmy = u * fay + v * fby + w * fcy
                nmz = u * faz + v * fbz + w * fcz

                nx = plsc.load_gather(mx_v.at[sl], [h16]) + (nmx * 0.5 - 0.25)
                ny = plsc.load_gather(my_v.at[sl], [h16]) + (nmy * 0.5 - 0.25)
                nz = plsc.load_gather(mz_v.at[sl], [h16]) + (nmz * 0.5 - 0.25)
                rn = _rsqrt(nx * nx + ny * ny + nz * nz)
                plsc.store_scatter(ox_v, [h16], nx * rn)
                plsc.store_scatter(oy_v, [h16], ny * rn)
                plsc.store_scatter(oz_v, [h16], nz * rn)
                return c2
            for q in range(NQ):
                for d in g2descs[q]:
                    d.wait()
                lax.fori_loop(GQ * q, GQ * (q + 1), compute, 0, unroll=2)

            pltpu.sync_copy(ox_v, ox_hbm.at[pl.ds(base, B)])
            pltpu.sync_copy(oy_v, oy_hbm.at[pl.ds(base, B)])
            pltpu.sync_copy(oz_v, oz_hbm.at[pl.ds(base, B)])
            return carry

        lax.fori_loop(0, NCHUNK, chunk_body, 0)

    return vertex_kernel


def kernel(hit_positions, hit_normals, hit_primIDs, vbo, ibo, features):
    H = hit_primIDs.shape[0]
    V = vbo.shape[0]
    T = ibo.shape[0]
    px, py, pz = (hit_positions[:, j] for j in range(3))
    nx, ny, nz = (hit_normals[:, j] for j in range(3))
    i0, i1, i2 = (ibo[:, j].reshape(T // 8, 8) for j in range(3))
    VP = 1 << 20
    pad = jnp.zeros((VP - V,), jnp.float32)
    cols = [jnp.concatenate([vbo[:, j], pad]) for j in range(3)]
    cols += [jnp.concatenate([features[:, j], pad]) for j in range(3)]
    comb = _make_interleave(VP, BV=4096)(*cols)
    k = _make_kernel(H, VP, T, B=1024)
    ox, oy, oz = k(px, py, pz, nx, ny, nz, hit_primIDs, i0, i1, i2, comb)
    return jnp.stack([ox, oy, oz], axis=1)
